# baseline (speedup 1.0000x reference)
import functools
import math

import jax
import jax.numpy as jnp
from jax.experimental import pallas as pl
from jax.experimental.pallas import tpu as pltpu

NUM_CLASSES = 8
NUM_ANCHORS = 3
NC5 = 5 + NUM_CLASSES
VMEM_LIMIT = 100 * 1024 * 1024


def _halo_conv1_weights(w):
    zero = jnp.zeros((12, 64), w.dtype)
    rows = []
    for ki in range(3):
        for vv in range(6):
            cols = []
            for q4 in range(4):
                kj = vv - q4
                cols.append(w[ki * 3 + kj] if 0 <= kj < 3 else zero)
            rows.append(jnp.concatenate(cols, axis=1))
    return jnp.concatenate(rows, axis=0)


def _pair_conv_weights(w):
    cin, cout = w.shape[1], w.shape[2]
    zero = jnp.zeros((cin, cout), w.dtype)
    taps = []
    for di in range(3):
        for dvi in range(3):
            rows = []
            for vp in range(2):
                cols = []
                for q in range(2):
                    dj = 2 * (dvi - 1) + vp + 1 - q
                    cols.append(w[di * 3 + dj] if 0 <= dj < 3 else zero)
                rows.append(jnp.concatenate(cols, axis=1))
            taps.append(jnp.concatenate(rows, axis=0))
    return jnp.stack(taps)


def _head_select(w, b, t, nq):
    cin = w.shape[0]
    obj_cols = jnp.array([a * NC5 + 4 for a in range(NUM_ANCHORS)], jnp.int32)
    cls_cols = jnp.array([a * NC5 + 5 for a in range(NUM_ANCHORS)], jnp.int32) + t
    wobj = jnp.take(w, obj_cols, axis=1).astype(jnp.bfloat16)
    wcls = jnp.take(w, cls_cols, axis=1).astype(jnp.bfloat16)
    ws = jnp.zeros((nq * cin, 128), jnp.bfloat16)
    bs = jnp.full((1, 128), -30.0, jnp.float32)
    for q in range(nq):
        ws = ws.at[q * cin:(q + 1) * cin, q * 16:q * 16 + 3].set(wobj)
        ws = ws.at[q * cin:(q + 1) * cin, 64 + q * 16:64 + q * 16 + 3].set(wcls)
        bs = bs.at[0, q * 16:q * 16 + 3].set(jnp.take(b[0], obj_cols))
        bs = bs.at[0, 64 + q * 16:64 + q * 16 + 3].set(jnp.take(b[0], cls_cols))
    return ws, bs


def _sigmoid(z):
    return 1.0 / (1.0 + jnp.exp(-z))


def _mega_kernel(xq_ref, w1_ref, b1_ref, h1w_ref, h1b_ref,
                 w2_ref, b2_ref, h2w_ref, h2b_ref,
                 w3_ref, b3_ref, h3w_ref, h3b_ref,
                 smax_ref, f1p_ref, f2p_ref,
                 *, h, w, nb):
    hr1, wq1 = h // 2, w // 8
    hr2, wp2 = h // 4, w // 8
    hr3, wp3 = h // 8, w // 16

    f1p_ref[pl.ds(0, 1), :, :] = jnp.zeros((1, wp2 + 2, 512), jnp.bfloat16)
    f1p_ref[pl.ds(hr2 + 1, 1), :, :] = jnp.zeros((1, wp2 + 2, 512), jnp.bfloat16)
    f1p_ref[:, pl.ds(0, 1), :] = jnp.zeros((hr2 + 2, 1, 512), jnp.bfloat16)
    f1p_ref[:, pl.ds(wp2 + 1, 1), :] = jnp.zeros((hr2 + 2, 1, 512), jnp.bfloat16)
    f2p_ref[pl.ds(0, 1), :, :] = jnp.zeros((1, wp3 + 2, 1024), jnp.bfloat16)
    f2p_ref[pl.ds(hr3 + 1, 1), :, :] = jnp.zeros((1, wp3 + 2, 1024), jnp.bfloat16)
    f2p_ref[:, pl.ds(0, 1), :] = jnp.zeros((hr3 + 2, 1, 1024), jnp.bfloat16)
    f2p_ref[:, pl.ds(wp3 + 1, 1), :] = jnp.zeros((hr3 + 2, 1, 1024), jnp.bfloat16)

    th1 = min(128, hr1)
    n1 = hr1 // th1

    for bb in range(nb):

      def level1(c, m, bb=bb):
        r0 = c * th1
        pieces = [xq_ref[bb, pl.ds(r0 + ki, th1), :, :] for ki in range(3)]
        patch = jnp.concatenate(pieces, axis=-1).reshape(th1 * wq1, 216)
        y = jnp.dot(patch, w1_ref[...], preferred_element_type=jnp.float32)
        y = y + b1_ref[...]
        y = y * _sigmoid(y)
        ybf = y.astype(jnp.bfloat16)
        z = jnp.dot(ybf, h1w_ref[...], preferred_element_type=jnp.float32)
        s = _sigmoid(z + h1b_ref[...])
        p = s[:, 0:64] * s[:, 64:128]
        m = jnp.maximum(m, jnp.max(p))
        yv = ybf.reshape(th1 // 2, 2, wq1, 256)
        for par in range(2):
            f1p_ref[pl.ds(1 + r0 // 2, th1 // 2), pl.ds(1, wq1),
                    pl.ds(par * 128, 128)] = yv[:, par, :, 0:128]
            f1p_ref[pl.ds(1 + r0 // 2, th1 // 2), pl.ds(1, wq1),
                    pl.ds(256 + par * 128, 128)] = yv[:, par, :, 128:256]
        return m

      m = jax.lax.fori_loop(0, n1, level1, jnp.zeros((), jnp.float32))

      th2 = min(32, hr2)
      n2 = hr2 // th2

      def level2(c, m):
        i0 = c * th2
        sh = [f1p_ref[pl.ds(i0, th2 + 2), pl.ds(dvi, wp2), :] for dvi in range(3)]
        acc = jnp.zeros((th2 * wp2, 256), jnp.float32)
        for di in range(3):
            for dvi in range(3):
                patch = sh[dvi][di:di + th2].reshape(th2 * wp2, 512)
                acc = acc + jnp.dot(patch, w2_ref[di * 3 + dvi],
                                    preferred_element_type=jnp.float32)
        y = acc + b2_ref[...]
        y = y * _sigmoid(y)
        ybf = y.astype(jnp.bfloat16)
        z = jnp.dot(ybf, h2w_ref[...], preferred_element_type=jnp.float32)
        s = _sigmoid(z + h2b_ref[...])
        p = s[:, 0:64] * s[:, 64:128]
        m = jnp.maximum(m, jnp.max(p))
        yv = ybf.reshape(th2 // 2, 2, wp2 // 2, 2, 256)
        for par in range(2):
            for vp in range(2):
                f2p_ref[pl.ds(1 + i0 // 2, th2 // 2), pl.ds(1, wp2 // 2),
                        pl.ds(vp * 512 + par * 256, 256)] = yv[:, par, :, vp, :]
        return m

      m = jax.lax.fori_loop(0, n2, level2, m)

      th3 = min(32, hr3)
      for c in range(hr3 // th3):
        i0 = c * th3
        sh = [f2p_ref[pl.ds(i0, th3 + 2), pl.ds(dvi, wp3), :] for dvi in range(3)]
        acc = jnp.zeros((th3 * wp3, 256), jnp.float32)
        for di in range(3):
            for dvi in range(3):
                patch = sh[dvi][di:di + th3].reshape(th3 * wp3, 1024)
                acc = acc + jnp.dot(patch, w3_ref[di * 3 + dvi],
                                    preferred_element_type=jnp.float32)
        y = acc + b3_ref[...]
        y = y * _sigmoid(y)
        ybf = y.astype(jnp.bfloat16)
        z = jnp.dot(ybf, h3w_ref[...], preferred_element_type=jnp.float32)
        s = _sigmoid(z + h3b_ref[...])
        p = s[:, 0:64] * s[:, 64:128]
        m = jnp.maximum(m, jnp.max(p))

      smax_ref[pl.ds(bb, 1), :, :] = m * jnp.ones((1, 1, 128), jnp.float32)


def _loss_kernel(sm_ref, loss_ref):
    m = jnp.max(sm_ref[...])
    loss_ref[...] = -jnp.log(jnp.maximum(1.0 - m, 1e-9)) * jnp.ones_like(loss_ref)


def _space_to_depth(x):
    b, h, w, c = x.shape
    x = x.reshape(b, h // 2, 2, w // 2, 2, c)
    x = jnp.transpose(x, (0, 1, 3, 2, 4, 5))
    return x.reshape(b, h // 2, w // 2, 4 * c)


def kernel(x, attack_target, conv1_w, conv1_b, conv2_w, conv2_b, conv3_w,
           conv3_b, head1_w, head1_b, head2_w, head2_b, head3_w, head3_b):
    t = jnp.asarray(attack_target, jnp.int32)
    x = jnp.transpose(x.astype(jnp.bfloat16), (0, 2, 3, 1))
    bsz, h, w, _ = x.shape

    xs = _space_to_depth(x)
    x48 = jnp.pad(xs, ((0, 0), (0, 0), (1, 3), (0, 0)))
    x48 = x48.reshape(bsz, h // 2, w // 8 + 1, 48)
    xq = jnp.concatenate(
        [x48[:, :, 0:w // 8, :], x48[:, :, 1:w // 8 + 1, 0:24]], axis=-1)
    xq = jnp.pad(xq, ((0, 0), (1, 1), (0, 0), (0, 0)))

    w1 = _halo_conv1_weights(conv1_w)
    w2 = _pair_conv_weights(conv2_w)
    w3 = _pair_conv_weights(conv3_w)
    b1 = jnp.tile(conv1_b, (1, 4))
    b2 = jnp.tile(conv2_b, (1, 2))
    b3 = jnp.tile(conv3_b, (1, 2))
    h1w, h1b = _head_select(head1_w, head1_b, t, 4)
    h2w, h2b = _head_select(head2_w, head2_b, t, 2)
    h3w, h3b = _head_select(head3_w, head3_b, t, 2)

    nb = 2 if bsz % 2 == 0 else 1
    const = lambda bi: (0, 0)
    const3 = lambda bi: (0, 0, 0)
    sm = pl.pallas_call(
        functools.partial(_mega_kernel, h=h, w=w, nb=nb),
        grid=(bsz // nb,),
        in_specs=[
            pl.BlockSpec((nb, h // 2 + 2, w // 8, 72),
                         lambda bi: (bi, 0, 0, 0)),
            pl.BlockSpec((216, 256), const),
            pl.BlockSpec((1, 256), const),
            pl.BlockSpec((256, 128), const),
            pl.BlockSpec((1, 128), const),
            pl.BlockSpec((9, 512, 256), const3),
            pl.BlockSpec((1, 256), const),
            pl.BlockSpec((256, 128), const),
            pl.BlockSpec((1, 128), const),
            pl.BlockSpec((9, 1024, 256), const3),
            pl.BlockSpec((1, 256), const),
            pl.BlockSpec((256, 128), const),
            pl.BlockSpec((1, 128), const),
        ],
        out_specs=pl.BlockSpec((nb, 1, 128), lambda bi: (bi, 0, 0)),
        out_shape=jax.ShapeDtypeStruct((bsz, 1, 128), jnp.float32),
        scratch_shapes=[
            pltpu.VMEM((h // 4 + 2, w // 8 + 2, 512), jnp.bfloat16),
            pltpu.VMEM((h // 8 + 2, w // 16 + 2, 1024), jnp.bfloat16),
        ],
        compiler_params=pltpu.CompilerParams(
            dimension_semantics=("parallel",),
            vmem_limit_bytes=VMEM_LIMIT),
    )(xq, w1, b1, h1w, h1b, w2, b2, h2w, h2b, w3, b3, h3w, h3b)

    loss = pl.pallas_call(
        _loss_kernel,
        grid=(1,),
        in_specs=[pl.BlockSpec((bsz, 128), lambda i: (0, 0))],
        out_specs=pl.BlockSpec((1, 1), lambda i: (0, 0)),
        out_shape=jax.ShapeDtypeStruct((1, 1), jnp.float32),
    )(sm.reshape(bsz, 128))
    return loss[0, 0]


# speedup vs baseline: 1.0216x; 1.0216x over previous
import functools
import math

import jax
import jax.numpy as jnp
from jax.experimental import pallas as pl
from jax.experimental.pallas import tpu as pltpu

NUM_CLASSES = 8
NUM_ANCHORS = 3
NC5 = 5 + NUM_CLASSES
VMEM_LIMIT = 100 * 1024 * 1024


def _halo_conv1_weights(w):
    zero = jnp.zeros((12, 64), w.dtype)
    rows = []
    for ki in range(3):
        for vv in range(6):
            cols = []
            for q4 in range(4):
                kj = vv - q4
                cols.append(w[ki * 3 + kj] if 0 <= kj < 3 else zero)
            rows.append(jnp.concatenate(cols, axis=1))
    return jnp.concatenate(rows, axis=0)


def _pair_conv_weights(w):
    cin, cout = w.shape[1], w.shape[2]
    zero = jnp.zeros((cin, cout), w.dtype)
    taps = []
    for di in range(3):
        for dvi in range(3):
            rows = []
            for vp in range(2):
                cols = []
                for q in range(2):
                    dj = 2 * (dvi - 1) + vp + 1 - q
                    cols.append(w[di * 3 + dj] if 0 <= dj < 3 else zero)
                rows.append(jnp.concatenate(cols, axis=1))
            taps.append(jnp.concatenate(rows, axis=0))
    return jnp.stack(taps)


def _head_select(w, b, t, nq):
    cin = w.shape[0]
    obj_cols = jnp.array([a * NC5 + 4 for a in range(NUM_ANCHORS)], jnp.int32)
    cls_cols = jnp.array([a * NC5 + 5 for a in range(NUM_ANCHORS)], jnp.int32) + t
    wobj = jnp.take(w, obj_cols, axis=1).astype(jnp.bfloat16)
    wcls = jnp.take(w, cls_cols, axis=1).astype(jnp.bfloat16)
    ws = jnp.zeros((nq * cin, 128), jnp.bfloat16)
    bs = jnp.full((1, 128), -30.0, jnp.float32)
    for q in range(nq):
        ws = ws.at[q * cin:(q + 1) * cin, q * 16:q * 16 + 3].set(wobj)
        ws = ws.at[q * cin:(q + 1) * cin, 64 + q * 16:64 + q * 16 + 3].set(wcls)
        bs = bs.at[0, q * 16:q * 16 + 3].set(jnp.take(b[0], obj_cols))
        bs = bs.at[0, 64 + q * 16:64 + q * 16 + 3].set(jnp.take(b[0], cls_cols))
    return ws, bs


def _sigmoid(z):
    return 1.0 / (1.0 + jnp.exp(-z))


def _mega_kernel(xq_ref, w1_ref, b1_ref, h1w_ref, h1b_ref,
                 w2_ref, b2_ref, h2w_ref, h2b_ref,
                 w3_ref, b3_ref, h3w_ref, h3b_ref,
                 smax_ref, f1p_ref, f2p_ref,
                 *, h, w):
    hr1, wq1 = h // 2, w // 8
    hr2, wp2 = h // 4, w // 8
    hr3, wp3 = h // 8, w // 16

    f1p_ref[pl.ds(0, 1), :, :] = jnp.zeros((1, wp2 + 2, 512), jnp.bfloat16)
    f1p_ref[pl.ds(hr2 + 1, 1), :, :] = jnp.zeros((1, wp2 + 2, 512), jnp.bfloat16)
    f1p_ref[:, pl.ds(0, 1), :] = jnp.zeros((hr2 + 2, 1, 512), jnp.bfloat16)
    f1p_ref[:, pl.ds(wp2 + 1, 1), :] = jnp.zeros((hr2 + 2, 1, 512), jnp.bfloat16)
    f2p_ref[pl.ds(0, 1), :, :] = jnp.zeros((1, wp3 + 2, 1024), jnp.bfloat16)
    f2p_ref[pl.ds(hr3 + 1, 1), :, :] = jnp.zeros((1, wp3 + 2, 1024), jnp.bfloat16)
    f2p_ref[:, pl.ds(0, 1), :] = jnp.zeros((hr3 + 2, 1, 1024), jnp.bfloat16)
    f2p_ref[:, pl.ds(wp3 + 1, 1), :] = jnp.zeros((hr3 + 2, 1, 1024), jnp.bfloat16)

    th1 = min(128, hr1)
    n1 = hr1 // th1

    def level1(c, m):
        r0 = c * th1
        pieces = [xq_ref[pl.ds(r0 + ki, th1), :, :] for ki in range(3)]
        patch = jnp.concatenate(pieces, axis=-1).reshape(th1 * wq1, 216)
        y = jnp.dot(patch, w1_ref[...], preferred_element_type=jnp.float32)
        y = y + b1_ref[...]
        y = y * _sigmoid(y)
        ybf = y.astype(jnp.bfloat16)
        z = jnp.dot(ybf, h1w_ref[...], preferred_element_type=jnp.float32)
        s = _sigmoid(z + h1b_ref[...])
        p = s[:, 0:64] * s[:, 64:128]
        m = jnp.maximum(m, jnp.max(p))
        yv = ybf.reshape(th1 // 2, 2, wq1, 256)
        for par in range(2):
            f1p_ref[pl.ds(1 + r0 // 2, th1 // 2), pl.ds(1, wq1),
                    pl.ds(par * 128, 128)] = yv[:, par, :, 0:128]
            f1p_ref[pl.ds(1 + r0 // 2, th1 // 2), pl.ds(1, wq1),
                    pl.ds(256 + par * 128, 128)] = yv[:, par, :, 128:256]
        return m

    m = jax.lax.fori_loop(0, n1, level1, jnp.zeros((), jnp.float32))

    th2 = min(32, hr2)
    n2 = hr2 // th2

    def level2(c, m):
        i0 = c * th2
        sh = [f1p_ref[pl.ds(i0, th2 + 2), pl.ds(dvi, wp2), :] for dvi in range(3)]
        acc = jnp.zeros((th2 * wp2, 256), jnp.float32)
        for di in range(3):
            for dvi in range(3):
                patch = sh[dvi][di:di + th2].reshape(th2 * wp2, 512)
                acc = acc + jnp.dot(patch, w2_ref[di * 3 + dvi],
                                    preferred_element_type=jnp.float32)
        y = acc + b2_ref[...]
        y = y * _sigmoid(y)
        ybf = y.astype(jnp.bfloat16)
        z = jnp.dot(ybf, h2w_ref[...], preferred_element_type=jnp.float32)
        s = _sigmoid(z + h2b_ref[...])
        p = s[:, 0:64] * s[:, 64:128]
        m = jnp.maximum(m, jnp.max(p))
        yv = ybf.reshape(th2 // 2, 2, wp2 // 2, 2, 256)
        for par in range(2):
            for vp in range(2):
                f2p_ref[pl.ds(1 + i0 // 2, th2 // 2), pl.ds(1, wp2 // 2),
                        pl.ds(vp * 512 + par * 256, 256)] = yv[:, par, :, vp, :]
        return m

    m = jax.lax.fori_loop(0, n2, level2, m)

    th3 = min(32, hr3)
    for c in range(hr3 // th3):
        i0 = c * th3
        sh = [f2p_ref[pl.ds(i0, th3 + 2), pl.ds(dvi, wp3), :] for dvi in range(3)]
        acc = jnp.zeros((th3 * wp3, 256), jnp.float32)
        for di in range(3):
            for dvi in range(3):
                patch = sh[dvi][di:di + th3].reshape(th3 * wp3, 1024)
                acc = acc + jnp.dot(patch, w3_ref[di * 3 + dvi],
                                    preferred_element_type=jnp.float32)
        y = acc + b3_ref[...]
        y = y * _sigmoid(y)
        ybf = y.astype(jnp.bfloat16)
        z = jnp.dot(ybf, h3w_ref[...], preferred_element_type=jnp.float32)
        s = _sigmoid(z + h3b_ref[...])
        p = s[:, 0:64] * s[:, 64:128]
        m = jnp.maximum(m, jnp.max(p))

    smax_ref[...] = m * jnp.ones_like(smax_ref)


def _loss_kernel(sm_ref, loss_ref):
    m = jnp.max(sm_ref[...])
    loss_ref[...] = -jnp.log(jnp.maximum(1.0 - m, 1e-9)) * jnp.ones_like(loss_ref)


def _space_to_depth(x):
    b, h, w, c = x.shape
    x = x.reshape(b, h // 2, 2, w // 2, 2, c)
    x = jnp.transpose(x, (0, 1, 3, 2, 4, 5))
    return x.reshape(b, h // 2, w // 2, 4 * c)


def kernel(x, attack_target, conv1_w, conv1_b, conv2_w, conv2_b, conv3_w,
           conv3_b, head1_w, head1_b, head2_w, head2_b, head3_w, head3_b):
    t = jnp.asarray(attack_target, jnp.int32)
    x = jnp.transpose(x.astype(jnp.bfloat16), (0, 2, 3, 1))
    bsz, h, w, _ = x.shape

    xs = _space_to_depth(x)
    x48 = jnp.pad(xs, ((0, 0), (0, 0), (1, 3), (0, 0)))
    x48 = x48.reshape(bsz, h // 2, w // 8 + 1, 48)
    xq = jnp.concatenate(
        [x48[:, :, 0:w // 8, :], x48[:, :, 1:w // 8 + 1, 0:24]], axis=-1)
    xq = jnp.pad(xq, ((0, 0), (1, 1), (0, 0), (0, 0)))

    w1 = _halo_conv1_weights(conv1_w)
    w2 = _pair_conv_weights(conv2_w)
    w3 = _pair_conv_weights(conv3_w)
    b1 = jnp.tile(conv1_b, (1, 4))
    b2 = jnp.tile(conv2_b, (1, 2))
    b3 = jnp.tile(conv3_b, (1, 2))
    h1w, h1b = _head_select(head1_w, head1_b, t, 4)
    h2w, h2b = _head_select(head2_w, head2_b, t, 2)
    h3w, h3b = _head_select(head3_w, head3_b, t, 2)

    const = lambda bi: (0, 0)
    const3 = lambda bi: (0, 0, 0)
    sm = pl.pallas_call(
        functools.partial(_mega_kernel, h=h, w=w),
        grid=(bsz,),
        in_specs=[
            pl.BlockSpec((None, h // 2 + 2, w // 8, 72),
                         lambda bi: (bi, 0, 0, 0)),
            pl.BlockSpec((216, 256), const),
            pl.BlockSpec((1, 256), const),
            pl.BlockSpec((256, 128), const),
            pl.BlockSpec((1, 128), const),
            pl.BlockSpec((9, 512, 256), const3),
            pl.BlockSpec((1, 256), const),
            pl.BlockSpec((256, 128), const),
            pl.BlockSpec((1, 128), const),
            pl.BlockSpec((9, 1024, 256), const3),
            pl.BlockSpec((1, 256), const),
            pl.BlockSpec((256, 128), const),
            pl.BlockSpec((1, 128), const),
        ],
        out_specs=pl.BlockSpec((None, 1, 128), lambda bi: (bi, 0, 0)),
        out_shape=jax.ShapeDtypeStruct((bsz, 1, 128), jnp.float32),
        scratch_shapes=[
            pltpu.VMEM((h // 4 + 2, w // 8 + 2, 512), jnp.bfloat16),
            pltpu.VMEM((h // 8 + 2, w // 16 + 2, 1024), jnp.bfloat16),
        ],
        compiler_params=pltpu.CompilerParams(
            dimension_semantics=("parallel",),
            vmem_limit_bytes=VMEM_LIMIT),
    )(xq, w1, b1, h1w, h1b, w2, b2, h2w, h2b, w3, b3, h3w, h3b)

    loss = pl.pallas_call(
        _loss_kernel,
        grid=(1,),
        in_specs=[pl.BlockSpec((bsz, 128), lambda i: (0, 0))],
        out_specs=pl.BlockSpec((1, 1), lambda i: (0, 0)),
        out_shape=jax.ShapeDtypeStruct((1, 1), jnp.float32),
    )(sm.reshape(bsz, 128))
    return loss[0, 0]
